# baseline (speedup 1.0000x reference)
import functools

import jax
import jax.numpy as jnp
from jax import lax
from jax.experimental import pallas as pl
from jax.experimental.pallas import tpu as pltpu
from jax.experimental.pallas import tpu_sc as plsc

VOCAB = 1024
D = 64
BATCH = 4096
SEQ = 200
B = BATCH * SEQ

NC = 2
NS = 16
NW = NC * NS
B_PER_W = B // NW
CHUNK = 800
NCHUNK = B_PER_W // CHUNK


_mesh = plsc.VectorSubcoreMesh(
    core_axis_name="c", subcore_axis_name="s", num_cores=NC, num_subcores=NS
)


@functools.partial(
    pl.kernel,
    out_type=jax.ShapeDtypeStruct((B, D), jnp.float32),
    mesh=_mesh,
    scratch_types=[
        pltpu.VMEM((2, CHUNK), jnp.int32),
        pltpu.VMEM((2, CHUNK, D), jnp.float32),
        pltpu.SemaphoreType.DMA,
        pltpu.SemaphoreType.DMA,
        pltpu.SemaphoreType.DMA,
        pltpu.SemaphoreType.DMA,
    ],
    compiler_params=pltpu.CompilerParams(use_tc_tiling_on_sc=False),
)
def _gather_kernel(table_hbm, idx_hbm, out_hbm, idx_v, rows_v, g0, g1, s0, s1):
    wid = lax.axis_index("s") * NC + lax.axis_index("c")
    base = wid * B_PER_W
    gsem = (g0, g1)
    ssem = (s0, s1)

    def load_and_gather(ci, slot):
        off = base + ci * CHUNK
        pltpu.sync_copy(idx_hbm.at[pl.ds(off, CHUNK)], idx_v.at[slot])
        pltpu.make_async_copy(
            table_hbm.at[idx_v.at[slot]], rows_v.at[slot], gsem[slot]
        ).start()

    def retire_gather_start_store(ci, slot):
        off = base + ci * CHUNK
        pltpu.make_async_copy(
            table_hbm.at[idx_v.at[slot]], rows_v.at[slot], gsem[slot]
        ).wait()
        pltpu.make_async_copy(
            rows_v.at[slot], out_hbm.at[pl.ds(off, CHUNK)], ssem[slot]
        ).start()

    def wait_store(ci, slot):
        off = base + ci * CHUNK
        pltpu.make_async_copy(
            rows_v.at[slot], out_hbm.at[pl.ds(off, CHUNK)], ssem[slot]
        ).wait()

    load_and_gather(0, 0)
    load_and_gather(1, 1)
    retire_gather_start_store(0, 0)
    retire_gather_start_store(1, 1)

    @pl.loop(1, NCHUNK // 2)
    def _pair(j):
        a = 2 * j
        wait_store(a - 2, 0)
        load_and_gather(a, 0)
        wait_store(a - 1, 1)
        load_and_gather(a + 1, 1)
        retire_gather_start_store(a, 0)
        retire_gather_start_store(a + 1, 1)

    wait_store(NCHUNK - 2, 0)
    wait_store(NCHUNK - 1, 1)


def kernel(ids, emb_table):
    flat = ids.reshape(B).astype(jnp.int32)
    out = _gather_kernel(emb_table, flat)
    return out.reshape(BATCH, SEQ, D)
